# baseline (speedup 1.0000x reference)
import jax
import jax.numpy as jnp
from jax.experimental import pallas as pl

B = 1024
N1, N2 = 25, 10
DIN = 128
H0, H1 = 256, 128
NODES = 1 + N1 + N1 * N2
BB = 64
PAD = 32


def _leaky(x):
    return jnp.where(x >= 0, x, x * 0.01)


def _tower(f, w1, b1, w2, b2):
    h32 = f[:, 0:PAD, :]
    parts = [jnp.mean(f[:, 1:1 + N1, :], axis=1, keepdims=True)]
    for j in range(N1):
        lo = 1 + N1 + N2 * j
        parts.append(jnp.mean(f[:, lo:lo + N2, :], axis=1, keepdims=True))
    parts.append(jnp.zeros((BB, PAD - 1 - N1, DIN), jnp.float32))
    n32 = jnp.concatenate(parts, axis=1)

    x = jnp.concatenate([h32, n32], axis=-1)
    l1 = _leaky(
        jnp.dot(x.reshape(BB * PAD, 2 * DIN), w1,
                preferred_element_type=jnp.float32)
        + b1
    ).reshape(BB, PAD, H0)

    h0n = l1[:, 0, :]
    neigh = jnp.mean(l1[:, 1:1 + N1, :], axis=1)
    h0f = _leaky(
        jnp.dot(jnp.concatenate([h0n, neigh], axis=-1), w2,
                preferred_element_type=jnp.float32)
        + b2
    )
    return _leaky(h0f)


def _fused_kernel(uf_ref, if_ref, w1u_ref, b1u_ref, w2u_ref, b2u_ref,
                  w1i_ref, b1i_ref, w2i_ref, b2i_ref,
                  wl_ref, bl_ref, out_ref):
    uh = _tower(uf_ref[...], w1u_ref[...], b1u_ref[...],
                w2u_ref[...], b2u_ref[...])
    ih = _tower(if_ref[...], w1i_ref[...], b1i_ref[...],
                w2i_ref[...], b2i_ref[...])
    p = uh * ih
    z = jnp.dot(p, wl_ref[...], preferred_element_type=jnp.float32) + bl_ref[...]
    out_ref[...] = jax.nn.sigmoid(z)


def kernel(sampling_user_feat, sampling_item_feat, W1_u, b1_u, W2_u, b2_u,
           W1_i, b1_i, W2_i, b2_i, W_lin, b_lin):
    b1u = b1_u.reshape(1, H0)
    b2u = b2_u.reshape(1, H1)
    b1i = b1_i.reshape(1, H0)
    b2i = b2_i.reshape(1, H1)
    wl = jnp.zeros((H1, 8), jnp.float32).at[:, :2].set(W_lin)
    bl = jnp.zeros((1, 8), jnp.float32).at[:, :2].set(b_lin)

    grid = B // BB
    feat_spec = pl.BlockSpec((BB, NODES, DIN), lambda i: (i, 0, 0))

    def wspec(shape):
        return pl.BlockSpec(shape, lambda i: tuple(0 for _ in shape))

    out = pl.pallas_call(
        _fused_kernel,
        grid=(grid,),
        in_specs=[
            feat_spec, feat_spec,
            wspec((2 * DIN, H0)), wspec((1, H0)),
            wspec((2 * H0, H1)), wspec((1, H1)),
            wspec((2 * DIN, H0)), wspec((1, H0)),
            wspec((2 * H0, H1)), wspec((1, H1)),
            wspec((H1, 8)), wspec((1, 8)),
        ],
        out_specs=pl.BlockSpec((BB, 8), lambda i: (i, 0)),
        out_shape=jax.ShapeDtypeStruct((B, 8), jnp.float32),
    )(sampling_user_feat, sampling_item_feat,
      W1_u, b1u, W2_u, b2u, W1_i, b1i, W2_i, b2i, wl, bl)
    return out[:, :2]
